# Initial kernel scaffold; baseline (speedup 1.0000x reference)
import jax
import jax.numpy as jnp
from jax.experimental import pallas as pl


def kernel(origins, directions, nears, fars, W1, b1, W2, b2):
    raise NotImplementedError("write your pallas kernel here")


# speedup vs baseline: 1.0036x; 1.0036x over previous
import functools

import numpy as np
import jax
import jax.numpy as jnp
from jax import lax
from jax.experimental import pallas as pl
from jax.experimental.pallas import tpu as pltpu
from jax.experimental.pallas import tpu_sc as plsc

NC = 2
NS = 16
NW = NC * NS

NSAMP = 128
NIMP = 64
STEPS = 4
BASE_VAR = 64.0
HID = 64
PAD = 3.0e38

_STEPS8 = (128, 64, 32, 16, 8, 4, 2, 1)
_STEPS5 = (16, 8, 4, 2, 1)

_U = (np.linspace(0.0, 1.0 - 1.0 / 17.0, 17) + 1.0 / 34.0).astype(np.float32)


def _take16(x, idx):
    return x.at[idx].get(mode="promise_in_bounds")


def _count_le(ref, x, steps, cap, strict):
    cnt = jnp.zeros((16,), jnp.int32)
    for s in steps:
        idx = jnp.minimum(cnt + (s - 1), cap)
        g = plsc.load_gather(ref, [idx])
        ok = (g < x) if strict else (g <= x)
        cnt = cnt + jnp.where(ok, s, 0)
    return cnt


def _sigmoid(x):
    return 1.0 / (1.0 + jnp.exp(-x))


def _round_bf16(x):
    u = plsc.bitcast(x, jnp.uint32)
    r = jax.lax.shift_right_logical(u, jnp.uint32(16)) & jnp.uint32(1)
    u = u + jnp.uint32(0x7FFF) + r
    u = u & jnp.uint32(0xFFFF0000)
    return plsc.bitcast(u, jnp.float32)


def _make(R):
    assert R % (NW * 8) == 0, R
    RPW = R // NW
    RB = 8
    NCH = RPW // RB

    f32 = jnp.float32
    out_type = (
        jax.ShapeDtypeStruct((R * NSAMP * 3,), f32),
        jax.ShapeDtypeStruct((R * NSAMP,), f32),
        jax.ShapeDtypeStruct((R * NIMP * 3,), f32),
        jax.ShapeDtypeStruct((R * 192 * 3,), f32),
    )
    scratch = [
        pltpu.VMEM((RPW * 8 + 8,), f32),
        pltpu.VMEM((192,), f32),
        pltpu.VMEM((144,), f32),
        pltpu.VMEM((384,), f32),
        pltpu.VMEM((256,), f32),
        pltpu.VMEM((256,), f32),
        pltpu.VMEM((208,), f32),
        pltpu.VMEM((208,), f32),
        pltpu.VMEM((256,), f32),
        pltpu.VMEM((192,), f32),
        pltpu.VMEM((128,), f32),
        pltpu.VMEM((32,), f32),
        pltpu.VMEM((RB * NSAMP * 3,), f32),
        pltpu.VMEM((RB * NSAMP,), f32),
        pltpu.VMEM((RB * NIMP * 3,), f32),
        pltpu.VMEM((RB * 192 * 3,), f32),
    ]

    def body(rayh, w1h, wvh, o_ip, o_iw, o_nsp, o_fin,
             inray, w1v, wvec, pbf,
             binsA, binsB, sdfA, sdfB, cdf, wref, epts, nbref,
             oip, oiw, onsp, ofin):
        wid = lax.axis_index("s") * NC + lax.axis_index("c")
        rbase = wid * RPW

        pltpu.sync_copy(rayh.at[pl.ds(rbase * 8, RPW * 8)],
                        inray.at[pl.ds(0, RPW * 8)])
        pltpu.sync_copy(w1h, w1v)
        pltpu.sync_copy(wvh, wvec)

        iota = lax.iota(jnp.int32, 16)
        iota_f = iota.astype(f32)
        idx15 = jnp.full((16,), 15, jnp.int32)
        lane0 = iota == 0
        padv = jnp.full((16,), PAD, f32)
        ones16 = jnp.ones((16,), f32)
        uA = iota_f * np.float32(1.0 / 17.0) + np.float32(1.0 / 34.0)
        uB = jnp.full((16,), float(_U[16]), f32)

        def bcast15(x):
            return _take16(x, idx15)

        def prodscan(q):
            y = q
            for s in (1, 2, 4, 8):
                g = _take16(y, jnp.maximum(iota - s, 0))
                y = y * jnp.where(iota >= s, g, 1.0)
            return y

        def ray_body(rl, ci, _):
            r = ci * RB + rl
            rd = inray[pl.ds(r * 8, 16)]
            ox, oy, oz = rd[0], rd[1], rd[2]
            dx, dy, dz = rd[3], rd[4], rd[5]
            near = rd[6]
            diff = rd[7] - near
            b2s = wvec[pl.ds(128, 16)][0]
            ovals = (ox, oy, oz)
            dvals = (dx, dy, dz)

            def initA(v, _c):
                g = iota + 16 * v
                binsA[pl.ds(16 * v, 16)] = jnp.where(
                    g <= 128, g.astype(f32) * (1.0 / 128.0), PAD)
                return 0
            lax.fori_loop(0, 16, initA, 0)

            def initB(v, _c):
                binsB[pl.ds(144 + 16 * v, 16)] = padv
                return 0
            lax.fori_loop(0, 7, initB, 0)

            def initC(v, _c):
                cdf[pl.ds(128 + 16 * v, 16)] = padv
                return 0
            lax.fori_loop(0, 8, initC, 0)
            cdf[pl.ds(0, 16)] = jnp.where(lane0, 0.0, PAD)
            nbref[pl.ds(16, 16)] = padv

            binsS, binsD = binsA, binsB
            sdfS, sdfD = sdfA, sdfB

            for it in range(STEPS):
                Lk = NSAMP + 16 * it
                nv = Lk // 16
                inv_s = BASE_VAR * (2.0 ** it)

                if it == 0:
                    def ept_body(v, _c):
                        e = near + binsS[pl.ds(16 * v, 16)] * diff
                        epts[pl.ds(16 * v, 16)] = e
                        pbf[pl.ds(16 * v, 16)] = _round_bf16(ox + dx * e)
                        pbf[pl.ds(128 + 16 * v, 16)] = _round_bf16(oy + dy * e)
                        pbf[pl.ds(256 + 16 * v, 16)] = _round_bf16(oz + dz * e)
                        return 0
                    lax.fori_loop(0, 8, ept_body, 0)

                    def mlp_body(k, acc):
                        w1x = w1v[pl.ds(16 * k, 16)]
                        w1y = w1v[pl.ds(64 + 16 * k, 16)]
                        w1z = w1v[pl.ds(128 + 16 * k, 16)]
                        b1g = wvec[pl.ds(16 * k, 16)]
                        w2g = wvec[pl.ds(64 + 16 * k, 16)]
                        pxs = [pbf[pl.ds(16 * v, 16)] for v in range(8)]
                        pys = [pbf[pl.ds(128 + 16 * v, 16)] for v in range(8)]
                        pzs = [pbf[pl.ds(256 + 16 * v, 16)] for v in range(8)]
                        acc = list(acc)
                        for j in range(16):
                            wx, wy, wz = w1x[j], w1y[j], w1z[j]
                            bh, wh = b1g[j], w2g[j]
                            for v in range(8):
                                m = (pxs[v] * wx + pys[v] * wy) + pzs[v] * wz + bh
                                acc[v] = acc[v] + _round_bf16(
                                    jnp.maximum(m, 0.0)) * wh
                        return tuple(acc)
                    z16 = jnp.zeros((16,), f32)
                    acc = lax.fori_loop(0, 4, mlp_body, (z16,) * 8)
                    for v in range(8):
                        sdfS[pl.ds(16 * v, 16)] = acc[v] + b2s

                    def ip_body(v, _c):
                        e = epts[pl.ds(16 * v, 16)]
                        pidx = rl * (NSAMP * 3) + (iota + 16 * v) * 3
                        for c in range(3):
                            plsc.store_scatter(
                                oip, [pidx + c], ovals[c] + dvals[c] * e)
                        return 0
                    lax.fori_loop(0, 8, ip_body, 0)

                def p2(i, car):
                    cp, ccos, wacc = car
                    b0 = binsS[pl.ds(16 * i, 16)]
                    b1_ = binsS[pl.ds(16 * i + 1, 16)]
                    s0 = sdfS[pl.ds(16 * i, 16)]
                    s1 = sdfS[pl.ds(16 * i + 1, 16)]
                    delta = (b1_ - b0) * diff
                    mid = 0.5 * (s0 + s1)
                    cos = (s1 - s0) / (delta + 1e-5)
                    sh = _take16(cos, jnp.maximum(iota - 1, 0))
                    pcos = jnp.where(lane0, ccos, sh)
                    c = jnp.clip(jnp.minimum(pcos, cos), -1000.0, 0.0)
                    hw = c * delta * 0.5
                    pc = _sigmoid((mid - hw) * inv_s)
                    nc = _sigmoid((mid + hw) * inv_s)
                    alpha = (pc - nc + 1e-5) / (pc + 1e-5)
                    q = 1.0 - alpha + 1e-7
                    incl = prodscan(q)
                    excl = jnp.where(iota >= 1, _take16(incl, jnp.maximum(iota - 1, 0)), 1.0)
                    w = alpha * (cp * excl)
                    w = jnp.where(iota + 16 * i >= Lk - 1, 0.0, w)
                    wref[pl.ds(16 * i, 16)] = w
                    if it == 0:
                        oiw[pl.ds(rl * NSAMP + 16 * i, 16)] = w
                    return (cp * bcast15(incl), bcast15(cos), wacc + w)
                _, _, accv = lax.fori_loop(
                    0, nv, p2, (ones16, jnp.zeros((16,), f32),
                                jnp.zeros((16,), f32)))
                total = bcast15(plsc.cumsum(accv))
                padw = jnp.maximum(1e-5 - total, 0.0)
                winc = padw / float(Lk)
                invw = 1.0 / (total + padw)

                def p4(i, cb):
                    pv = (wref[pl.ds(16 * i, 16)] + winc) * invw
                    incl = plsc.cumsum(pv)
                    cdf[pl.ds(16 * i + 1, 16)] = jnp.minimum(1.0, cb + incl)
                    return cb + bcast15(incl)
                lax.fori_loop(0, nv, p4, jnp.zeros((16,), f32))

                def sample(u):
                    cnt = _count_le(cdf, u, _STEPS8, 255, False)
                    below = cnt - 1
                    above = jnp.minimum(cnt, Lk)
                    cg0 = plsc.load_gather(cdf, [below])
                    cg1 = plsc.load_gather(cdf, [above])
                    bg0 = plsc.load_gather(binsS, [below])
                    bg1 = plsc.load_gather(binsS, [above])
                    denom = cg1 - cg0
                    denom = jnp.where(denom < 1e-5, 1.0, denom)
                    t = jnp.clip((u - cg0) / denom, 0.0, 1.0)
                    return bg0 + t * (bg1 - bg0)
                nbA = sample(uA)
                nbB = sample(uB)
                nbref[pl.ds(0, 16)] = nbA
                ev = near + nbA * diff
                pidx = rl * (NIMP * 3) + (it * 16 + iota) * 3
                for c in range(3):
                    plsc.store_scatter(onsp, [pidx + c], ovals[c] + dvals[c] * ev)

                p16x = _round_bf16(ox + dx * ev)
                p16y = _round_bf16(oy + dy * ev)
                p16z = _round_bf16(oz + dz * ev)

                def mlp16(k, a):
                    w1x = w1v[pl.ds(16 * k, 16)]
                    w1y = w1v[pl.ds(64 + 16 * k, 16)]
                    w1z = w1v[pl.ds(128 + 16 * k, 16)]
                    b1g = wvec[pl.ds(16 * k, 16)]
                    w2g = wvec[pl.ds(64 + 16 * k, 16)]
                    for j in range(16):
                        m = ((p16x * w1x[j] + p16y * w1y[j]) + p16z * w1z[j]
                             + b1g[j])
                        a = a + _round_bf16(jnp.maximum(m, 0.0)) * w2g[j]
                    return a
                nsdf16 = lax.fori_loop(0, 4, mlp16, jnp.zeros((16,), f32)) + b2s

                def p7(i, _c):
                    av = binsS[pl.ds(16 * i, 16)]
                    cntb = _count_le(nbref, av, _STEPS5, 31, True)
                    pos = iota + 16 * i + cntb
                    plsc.store_scatter(binsD, [pos], av)
                    plsc.store_scatter(sdfD, [pos], sdfS[pl.ds(16 * i, 16)])
                    return 0
                lax.fori_loop(0, nv, p7, 0)

                cnta = _count_le(binsS, nbA, _STEPS8, 255, False)
                posB = iota + cnta
                plsc.store_scatter(binsD, [posB], nbA)
                plsc.store_scatter(sdfD, [posB], nsdf16)
                endv = jnp.maximum(binsS[pl.ds(Lk, 16)][0], nbB)
                plsc.store_scatter(binsD, [jnp.full((16,), Lk + 16, jnp.int32)],
                                   endv, mask=lane0)

                binsS, binsD = binsD, binsS
                sdfS, sdfD = sdfD, sdfS

            def fin_body(v, _c):
                e = near + binsS[pl.ds(16 * v, 16)] * diff
                pidx = rl * (192 * 3) + (iota + 16 * v) * 3
                for c in range(3):
                    plsc.store_scatter(ofin, [pidx + c], ovals[c] + dvals[c] * e)
                return 0
            lax.fori_loop(0, 12, fin_body, 0)
            return 0

        def chunk_body(ci, _c):
            lax.fori_loop(0, RB, lambda rl, c: ray_body(rl, ci, c), 0)
            base = rbase + ci * RB
            pltpu.sync_copy(oip, o_ip.at[pl.ds(base * NSAMP * 3, RB * NSAMP * 3)])
            pltpu.sync_copy(oiw, o_iw.at[pl.ds(base * NSAMP, RB * NSAMP)])
            pltpu.sync_copy(onsp, o_nsp.at[pl.ds(base * NIMP * 3, RB * NIMP * 3)])
            pltpu.sync_copy(ofin, o_fin.at[pl.ds(base * 192 * 3, RB * 192 * 3)])
            return 0
        lax.fori_loop(0, NCH, chunk_body, 0)

    return body, out_type, scratch


@functools.lru_cache(maxsize=2)
def _build(R):
    body, out_type, scratch = _make(R)
    mesh = plsc.VectorSubcoreMesh(core_axis_name="c", subcore_axis_name="s",
                                  num_cores=NC, num_subcores=NS)
    return pl.kernel(body, out_type=out_type, mesh=mesh, scratch_types=scratch,
                     compiler_params=pltpu.CompilerParams(
                         needs_layout_passes=False))


def kernel(origins, directions, nears, fars, W1, b1, W2, b2):
    R = origins.shape[0]
    f32 = jnp.float32
    def _rnd(x):
        u = jax.lax.bitcast_convert_type(x.astype(f32), jnp.uint32)
        r = jax.lax.shift_right_logical(u, np.uint32(16)) & np.uint32(1)
        u = (u + np.uint32(0x7FFF) + r) & np.uint32(0xFFFF0000)
        return jax.lax.bitcast_convert_type(u, f32)

    W1r = _rnd(W1)
    W2r = _rnd(W2)
    wvec = jnp.concatenate([
        b1.astype(f32).reshape(-1),
        W2r.reshape(-1),
        jnp.broadcast_to(b2.astype(f32).reshape(-1), (16,)),
    ])
    raydat = jnp.concatenate([
        origins.astype(f32), directions.astype(f32),
        nears.astype(f32).reshape(R, 1), fars.astype(f32).reshape(R, 1),
    ], axis=1).reshape(-1)
    fn = _build(R)
    ip, iw, nsp, fin = fn(raydat, W1r.reshape(-1), wvec)
    return (ip.reshape(R, NSAMP, 3), iw.reshape(R, NSAMP),
            nsp.reshape(R, NIMP, 3), fin.reshape(R, 192, 3))
